# baseline (speedup 1.0000x reference)
import jax
import jax.numpy as jnp
from jax.experimental import pallas as pl
from jax.experimental.pallas import tpu as pltpu


def _router_block(x_ref, wt_ref, b_ref, w_out_ref, i_out_ref, acc_ref):
    j = pl.program_id(1)
    nd = pl.num_programs(1)
    part = jax.lax.dot_general(
        x_ref[...], wt_ref[...],
        dimension_numbers=(((1,), (0,)), ((), ())),
        preferred_element_type=jnp.float32,
    )

    @pl.when(j == 0)
    def _init():
        acc_ref[...] = part + b_ref[...]

    @pl.when(j != 0)
    def _acc():
        acc_ref[...] += part

    @pl.when(j == nd - 1)
    def _epilogue():
        logits = acc_ref[...]
        m = jnp.max(logits, axis=1, keepdims=True)
        s = jnp.sum(jnp.exp(logits - m), axis=1)
        w_out_ref[0, 0, :] = 1.0 / s
        iota = jax.lax.broadcasted_iota(jnp.int32, logits.shape, 1)
        idx = jnp.min(jnp.where(logits == m, iota, logits.shape[1]), axis=1)
        i_out_ref[0, 0, :] = idx


def kernel(x, W, b):
    tokens, d_model = x.shape
    num_experts = W.shape[0]
    tile = min(2048, tokens)
    dchunk = min(2048, d_model)
    grid = (tokens // tile, d_model // dchunk)
    wt = W.T
    b2 = b.reshape(1, num_experts)
    weights, indices = pl.pallas_call(
        _router_block,
        grid=grid,
        in_specs=[
            pl.BlockSpec((tile, dchunk), lambda i, j: (i, j)),
            pl.BlockSpec((dchunk, num_experts), lambda i, j: (j, 0)),
            pl.BlockSpec((1, num_experts), lambda i, j: (0, 0)),
        ],
        out_specs=[
            pl.BlockSpec((1, 1, tile), lambda i, j: (i, 0, 0)),
            pl.BlockSpec((1, 1, tile), lambda i, j: (i, 0, 0)),
        ],
        out_shape=[
            jax.ShapeDtypeStruct((grid[0], 1, tile), jnp.float32),
            jax.ShapeDtypeStruct((grid[0], 1, tile), jnp.int32),
        ],
        scratch_shapes=[pltpu.VMEM((tile, num_experts), jnp.float32)],
        compiler_params=pltpu.CompilerParams(
            dimension_semantics=("parallel", "arbitrary"),
        ),
    )(x, wt, b2)
    return weights.reshape(tokens), indices.reshape(tokens)


# speedup vs baseline: 1.1073x; 1.1073x over previous
import jax
import jax.numpy as jnp
from jax.experimental import pallas as pl
from jax.experimental.pallas import tpu as pltpu

_NBUF = 6
_CHUNK = 512


def _router_body(x_hbm, wt_ref, b_ref, w_out_ref, i_out_ref, bufs, sems):
    nchunks = x_hbm.shape[0] // _CHUNK
    num_experts = wt_ref.shape[1]

    def _copy(c, slot):
        return pltpu.make_async_copy(
            x_hbm.at[pl.ds(c * _CHUNK, _CHUNK), :],
            bufs.at[slot],
            sems.at[slot],
        )

    for c in range(min(_NBUF, nchunks)):
        _copy(c, c).start()

    wt = wt_ref[...]
    bias = b_ref[...]

    def _step(c, carry):
        slot = jax.lax.rem(c, _NBUF)
        _copy(c, slot).wait()
        logits = jax.lax.dot_general(
            bufs[slot], wt,
            dimension_numbers=(((1,), (0,)), ((), ())),
            preferred_element_type=jnp.float32,
        ) + bias
        m = jnp.max(logits, axis=1, keepdims=True)
        s = jnp.sum(jnp.exp(logits - m), axis=1)
        w_out_ref[c, :] = 1.0 / s
        iota = jax.lax.broadcasted_iota(jnp.int32, logits.shape, 1)
        i_out_ref[c, :] = jnp.min(
            jnp.where(logits == m, iota, num_experts), axis=1)

        @pl.when(c + _NBUF < nchunks)
        def _prefetch():
            _copy(c + _NBUF, slot).start()

        return carry

    jax.lax.fori_loop(0, nchunks, _step, 0)


def kernel(x, W, b):
    tokens, d_model = x.shape
    num_experts = W.shape[0]
    nchunks = tokens // _CHUNK
    wt = W.T
    b2 = b.reshape(1, num_experts)
    weights, indices = pl.pallas_call(
        _router_body,
        in_specs=[
            pl.BlockSpec(memory_space=pl.ANY),
            pl.BlockSpec(memory_space=pltpu.VMEM),
            pl.BlockSpec(memory_space=pltpu.VMEM),
        ],
        out_specs=[
            pl.BlockSpec(memory_space=pltpu.VMEM),
            pl.BlockSpec(memory_space=pltpu.VMEM),
        ],
        out_shape=[
            jax.ShapeDtypeStruct((nchunks, _CHUNK), jnp.float32),
            jax.ShapeDtypeStruct((nchunks, _CHUNK), jnp.int32),
        ],
        scratch_shapes=[
            pltpu.VMEM((_NBUF, _CHUNK, d_model), jnp.float32),
            pltpu.SemaphoreType.DMA((_NBUF,)),
        ],
    )(x, wt, b2)
    return weights.reshape(tokens), indices.reshape(tokens)
